# Initial kernel scaffold; baseline (speedup 1.0000x reference)
import jax
import jax.numpy as jnp
from jax.experimental import pallas as pl


def kernel(probs, edge_batch):
    raise NotImplementedError("write your pallas kernel here")


# speedup vs baseline: 101.9294x; 101.9294x over previous
import functools

import jax
import jax.numpy as jnp
from jax import lax
from jax.experimental import pallas as pl
from jax.experimental.pallas import tpu as pltpu
from jax.experimental.pallas import tpu_sc as plsc

N = 1600000
G = 128
NC = 2
NS = 16
NW = NC * NS
SEGS_PER_W = G // NW
BIG = (1 << 30)


def _make(n, blk, stride):
    chunk = n // NW
    assert chunk % blk == 0 and blk % 16 == 0 and n % NW == 0
    assert n % stride == 0
    nsamp = n // stride
    nsamp_pad = ((nsamp + 15) // 16) * 16
    win = ((stride + 1 + 15) // 16) * 16
    pad = blk + win
    nvec = blk // 16

    mesh = plsc.VectorSubcoreMesh(core_axis_name="c", subcore_axis_name="s",
                                  num_cores=NC, num_subcores=NS)

    def params_body(bits_hbm, eb_hbm, samp_hbm, params_hbm,
                    samp_v, win_v, blk_v, hist_v, tot_v, csuf_v, row_v):
        lanes = lax.iota(jnp.int32, 16)
        wid = lax.axis_index("s") * NC + lax.axis_index("c")
        pltpu.sync_copy(samp_hbm, samp_v)

        def lower_bound(t):
            def cnt_body(i, acc):
                v = samp_v[pl.ds(i * 16, 16)]
                return acc + (v < t).astype(jnp.int32)
            accv = lax.fori_loop(0, nsamp_pad // 16, cnt_body,
                                 jnp.zeros(16, jnp.int32))
            j0 = jnp.sum(accv)
            w0 = pl.multiple_of(jnp.maximum(j0 - 1, 0) * stride, 8)
            pltpu.sync_copy(eb_hbm.at[pl.ds(w0, win)], win_v)

            def cnt2_body(i, acc):
                v = win_v[pl.ds(i * 16, 16)]
                pos = i * 16 + lanes
                m = (pos <= stride) & (v < t)
                return acc + m.astype(jnp.int32)
            accv2 = lax.fori_loop(0, win // 16, cnt2_body,
                                  jnp.zeros(16, jnp.int32))
            return w0 + jnp.sum(accv2)

        bounds = [lower_bound(SEGS_PER_W * wid + j)
                  for j in range(SEGS_PER_W + 1)]

        for j in range(SEGS_PER_W):
            g = SEGS_PER_W * wid + j
            s, e = bounds[j], bounds[j + 1]
            c = e - s
            m = (c - 1) >> 1
            k0 = jnp.where((c & 1) == 0, c >> 1, m + (m & 1))
            k = jnp.minimum(jnp.maximum(k0, 1), c)
            base = s & jnp.int32(-8)
            nblk = (c + (s - base) + blk - 1) // blk

            def radix_pass(p, carry):
                prefix, k_rem, _ = carry
                shift = 20 - 10 * p

                def zero_body(i, _):
                    hist_v[pl.ds(i * 16, 16)] = jnp.zeros(16, jnp.int32)
                    return 0
                lax.fori_loop(0, 1024, zero_body, 0)

                def blk_body(b, _):
                    off = pl.multiple_of(base + b * blk, 8)
                    pltpu.sync_copy(bits_hbm.at[pl.ds(off, blk)], blk_v)

                    def vec_body(i, _):
                        v = blk_v[pl.ds(i * 16, 16)]
                        gpos = off + i * 16 + lanes
                        valid = ((gpos >= s) & (gpos < e)
                                 & ((v >> (shift + 10)) == prefix))
                        dig = (v >> shift) & 1023
                        plsc.addupdate_scatter(
                            hist_v, [lanes * 1024 + dig],
                            jnp.ones(16, jnp.int32), mask=valid)
                        return 0
                    lax.fori_loop(0, nvec, vec_body, 0)
                    return 0
                lax.fori_loop(0, nblk, blk_body, 0)

                def red_body(ch, _):
                    def lane_body(l, acc):
                        return acc + hist_v[pl.ds(l * 1024 + ch * 16, 16)]
                    accv = lax.fori_loop(0, 16, lane_body,
                                         jnp.zeros(16, jnp.int32))
                    tot_v[pl.ds(ch * 16, 16)] = accv
                    return 0
                lax.fori_loop(0, 64, red_body, 0)

                def s_body(t, carry2):
                    running, accnt = carry2
                    jv = 63 - t
                    v = tot_v[pl.ds(jv * 16, 16)]
                    ssum = jnp.sum(v)
                    pe = plsc.cumsum(v) - v
                    cv = running + (ssum - pe)
                    csuf_v[pl.ds(jv * 16, 16)] = cv
                    accnt = accnt + (cv >= k_rem).astype(jnp.int32)
                    return (running + ssum, accnt)
                _, accnt = lax.fori_loop(
                    0, 64, s_body, (jnp.int32(0), jnp.zeros(16, jnp.int32)))
                dstar = jnp.sum(accnt) - 1
                didx = jnp.zeros(16, jnp.int32) + dstar
                totd = jnp.max(plsc.load_gather(tot_v, [didx]))
                cd = jnp.max(plsc.load_gather(csuf_v, [didx]))
                k_rem = k_rem - (cd - totd)
                prefix = (prefix << 10) | dstar
                return (prefix, k_rem, totd)

            T, r, E = lax.fori_loop(
                0, 3, radix_pass, (jnp.int32(0), k, jnp.int32(0)))

            def find_l(_):
                def blk_l(b, carry):
                    def vec_l(i, carry2):
                        mcnt, lv = carry2
                        off = pl.multiple_of(base + b * blk, 8)
                        v = blk_v[pl.ds(i * 16, 16)]
                        gpos = off + i * 16 + lanes
                        match = (gpos >= s) & (gpos < e) & (v == T)
                        mi = match.astype(jnp.int32)
                        inc = plsc.cumsum(mi)
                        hit = match & ((inc + mcnt) == r)
                        lv = jnp.maximum(lv, jnp.max(jnp.where(hit, gpos, -1)))
                        return (mcnt + jnp.sum(mi), lv)
                    off = pl.multiple_of(base + b * blk, 8)
                    pltpu.sync_copy(bits_hbm.at[pl.ds(off, blk)], blk_v)
                    return lax.fori_loop(0, nvec, vec_l, carry)
                _, lv = lax.fori_loop(0, nblk, blk_l,
                                      (jnp.int32(0), jnp.int32(-1)))
                return lv
            L = lax.cond(r < E, find_l, lambda _: jnp.int32(n), 0)

            tq = jnp.where(c > 0, T, jnp.int32(0x7FFFFFFF))
            lq = jnp.where(c > 0, L, jnp.int32(-1))
            row_v[...] = jnp.where(lanes == 0, tq,
                                   jnp.where(lanes == 1, lq, 0))
            pltpu.sync_copy(row_v, params_hbm.at[g])

    cparams = pltpu.CompilerParams(needs_layout_passes=False)

    phase1 = functools.partial(
        pl.kernel,
        out_type=jax.ShapeDtypeStruct((G, 16), jnp.int32),
        mesh=mesh,
        compiler_params=cparams,
        scratch_types=[
            pltpu.VMEM((nsamp_pad,), jnp.int32),
            pltpu.VMEM((win,), jnp.int32),
            pltpu.VMEM((blk,), jnp.int32),
            pltpu.VMEM((16 * 1024,), jnp.int32),
            pltpu.VMEM((1024,), jnp.int32),
            pltpu.VMEM((1024,), jnp.int32),
            pltpu.VMEM((16,), jnp.int32),
        ],
    )(params_body)

    def mask_body(bits_hbm, eb_hbm, params_hbm, y_hbm,
                  par_v, bits_v, eb_v, out_v):
        lanes = lax.iota(jnp.int32, 16)
        zer = lanes * 0
        wid = lax.axis_index("s") * NC + lax.axis_index("c")
        pltpu.sync_copy(params_hbm, par_v)
        start = wid * chunk

        def blk_body(b, _):
            off = pl.multiple_of(start + b * blk, 8)
            pltpu.sync_copy(bits_hbm.at[pl.ds(off, blk)], bits_v)
            pltpu.sync_copy(eb_hbm.at[pl.ds(off, blk)], eb_v)

            def vec_body(i, _):
                v = bits_v[pl.ds(i * 16, 16)]
                sg = eb_v[pl.ds(i * 16, 16)]
                tt = plsc.load_gather(par_v, [sg, zer])
                ll = plsc.load_gather(par_v, [sg, zer + 1])
                gpos = off + i * 16 + lanes
                sel = (v > tt) | ((v == tt) & (gpos <= ll))
                out_v[pl.ds(i * 16, 16)] = jnp.where(
                    sel, jnp.float32(1), jnp.float32(0))
                return 0
            lax.fori_loop(0, nvec, vec_body, 0)
            pltpu.sync_copy(out_v, y_hbm.at[pl.ds(off, blk)])
            return 0
        lax.fori_loop(0, chunk // blk, blk_body, 0)

    phase2 = functools.partial(
        pl.kernel,
        out_type=jax.ShapeDtypeStruct((n,), jnp.float32),
        mesh=mesh,
        compiler_params=cparams,
        scratch_types=[
            pltpu.VMEM((G, 16), jnp.int32),
            pltpu.VMEM((blk,), jnp.int32),
            pltpu.VMEM((blk,), jnp.int32),
            pltpu.VMEM((blk,), jnp.float32),
        ],
    )(mask_body)

    def run(probs, edge_batch):
        bits = lax.bitcast_convert_type(probs, jnp.int32)
        bits_pad = jnp.pad(bits, (0, pad))
        eb_pad = jnp.pad(edge_batch.astype(jnp.int32), (0, pad),
                         constant_values=BIG)
        samp = eb_pad[: n : stride]
        samp_pad = jnp.pad(samp, (0, nsamp_pad - nsamp), constant_values=BIG)
        params = phase1(bits_pad, eb_pad, samp_pad)
        return phase2(bits_pad, eb_pad, params)

    run.phase1 = phase1
    run.phase2 = phase2
    return run


_run = _make(N, blk=2000, stride=512)


def kernel(probs, edge_batch):
    return _run(probs, edge_batch)


# speedup vs baseline: 134.1584x; 1.3162x over previous
import functools

import jax
import jax.numpy as jnp
from jax import lax
from jax.experimental import pallas as pl
from jax.experimental.pallas import tpu as pltpu
from jax.experimental.pallas import tpu_sc as plsc

N = 1600000
G = 128
NC = 2
NS = 16
NW = NC * NS
SEGS_PER_W = G // NW
BIG = (1 << 30)
SENT = 0x40000000


def _make(n, blk, stride, cap):
    chunk = n // NW
    assert chunk % blk == 0 and blk % 16 == 0 and n % NW == 0
    assert n % stride == 0 and cap % blk == 0
    nsamp = n // stride
    nsamp_pad = ((nsamp + 15) // 16) * 16
    win = ((stride + 1 + 15) // 16) * 16
    pad = blk + win
    nvec = blk // 16

    mesh = plsc.VectorSubcoreMesh(core_axis_name="c", subcore_axis_name="s",
                                  num_cores=NC, num_subcores=NS)
    cparams = pltpu.CompilerParams(needs_layout_passes=False)

    def params_body(bits_hbm, eb_hbm, samp_hbm, params_hbm,
                    samp_v, win_v, big_v, hist_v, tot_v, csuf_v, row_v, dsem):
        lanes = lax.iota(jnp.int32, 16)
        wid = lax.axis_index("s") * NC + lax.axis_index("c")
        pltpu.sync_copy(samp_hbm, samp_v)

        def lower_bound(t):
            def cnt_body(i, acc):
                v = samp_v[pl.ds(i * 16, 16)]
                return acc + (v < t).astype(jnp.int32)
            accv = lax.fori_loop(0, nsamp_pad // 16, cnt_body,
                                 jnp.zeros(16, jnp.int32))
            j0 = jnp.sum(accv)
            w0 = pl.multiple_of(jnp.maximum(j0 - 1, 0) * stride, 8)
            pltpu.sync_copy(eb_hbm.at[pl.ds(w0, win)], win_v)

            def cnt2_body(i, acc):
                v = win_v[pl.ds(i * 16, 16)]
                pos = i * 16 + lanes
                m = (pos <= stride) & (v < t)
                return acc + m.astype(jnp.int32)
            accv2 = lax.fori_loop(0, win // 16, cnt2_body,
                                  jnp.zeros(16, jnp.int32))
            return w0 + jnp.sum(accv2)

        bounds = [lower_bound(SEGS_PER_W * wid + j)
                  for j in range(SEGS_PER_W + 1)]

        for j in range(SEGS_PER_W):
            g = SEGS_PER_W * wid + j
            s, e = bounds[j], bounds[j + 1]
            c = e - s
            m = (c - 1) >> 1
            k0 = jnp.where((c & 1) == 0, c >> 1, m + (m & 1))
            k = jnp.minimum(jnp.maximum(k0, 1), c)
            base = s & jnp.int32(-8)
            head = s - base
            c_adj = c + head
            nchunks = (c_adj + cap - 1) // cap

            def load_chunk(ch):
                clen = jnp.minimum(c_adj - ch * cap, cap)
                nb = (clen + blk - 1) // blk

                def fire(b, _):
                    off = pl.multiple_of(base + ch * cap + b * blk, 8)
                    pltpu.make_async_copy(
                        bits_hbm.at[pl.ds(off, blk)],
                        big_v.at[pl.ds(b * blk, blk)], dsem).start()
                    return 0
                lax.fori_loop(0, nb, fire, 0)

                def drain(b, _):
                    pltpu.make_async_copy(
                        bits_hbm.at[pl.ds(0, blk)],
                        big_v.at[pl.ds(0, blk)], dsem).wait()
                    return 0
                lax.fori_loop(0, nb, drain, 0)
                head_eff = jnp.where(ch == 0, head, 0)
                v0 = big_v[pl.ds(0, 16)]
                big_v[pl.ds(0, 16)] = jnp.where(lanes < head_eff, SENT, v0)
                nv = (clen + 15) // 16
                toff = (nv - 1) * 16
                vt = big_v[pl.ds(toff, 16)]
                big_v[pl.ds(toff, 16)] = jnp.where(
                    toff + lanes < clen, vt, SENT)
                return nv

            def radix_pass(p, carry):
                prefix, k_rem, _ = carry
                shift = 20 - 10 * p

                def zero_body(i, _):
                    hist_v[pl.ds(i * 16, 16)] = jnp.zeros(16, jnp.int32)
                    return 0
                lax.fori_loop(0, 1024, zero_body, 0)

                def chunk_body(ch, _):
                    clen = jnp.minimum(c_adj - ch * cap, cap)
                    nv = (clen + 15) // 16

                    @pl.when((p == 0) | (nchunks > 1))
                    def _():
                        load_chunk(ch)

                    def vec_body(i, _):
                        v = big_v[pl.ds(i * 16, 16)]
                        part = (v >> (shift + 10)) == prefix
                        dig = (v >> shift) & 1023
                        plsc.addupdate_scatter(
                            hist_v, [lanes * 1024 + dig],
                            jnp.ones(16, jnp.int32), mask=part)
                        return 0
                    lax.fori_loop(0, nv, vec_body, 0)
                    return 0
                lax.fori_loop(0, nchunks, chunk_body, 0)

                def red_body(chh, _):
                    def lane_body(l, acc):
                        return acc + hist_v[pl.ds(l * 1024 + chh * 16, 16)]
                    accv = lax.fori_loop(0, 16, lane_body,
                                         jnp.zeros(16, jnp.int32))
                    tot_v[pl.ds(chh * 16, 16)] = accv
                    return 0
                lax.fori_loop(0, 64, red_body, 0)

                def s_body(t, carry2):
                    running, accnt = carry2
                    jv = 63 - t
                    v = tot_v[pl.ds(jv * 16, 16)]
                    ssum = jnp.sum(v)
                    pe = plsc.cumsum(v) - v
                    cv = running + (ssum - pe)
                    csuf_v[pl.ds(jv * 16, 16)] = cv
                    accnt = accnt + (cv >= k_rem).astype(jnp.int32)
                    return (running + ssum, accnt)
                _, accnt = lax.fori_loop(
                    0, 64, s_body, (jnp.int32(0), jnp.zeros(16, jnp.int32)))
                dstar = jnp.sum(accnt) - 1
                didx = jnp.zeros(16, jnp.int32) + dstar
                totd = jnp.max(plsc.load_gather(tot_v, [didx]))
                cd = jnp.max(plsc.load_gather(csuf_v, [didx]))
                k_rem = k_rem - (cd - totd)
                prefix = (prefix << 10) | dstar
                return (prefix, k_rem, totd)

            T, r, E = lax.fori_loop(
                0, 3, radix_pass, (jnp.int32(0), k, jnp.int32(0)))

            def find_l(_):
                def chunk_l(ch, carry):
                    clen = jnp.minimum(c_adj - ch * cap, cap)
                    nv = (clen + 15) // 16

                    @pl.when(nchunks > 1)
                    def _():
                        load_chunk(ch)

                    def vec_l(i, carry2):
                        mcnt, lv = carry2
                        v = big_v[pl.ds(i * 16, 16)]
                        gpos = base + ch * cap + i * 16 + lanes
                        match = v == T
                        mi = match.astype(jnp.int32)
                        inc = plsc.cumsum(mi)
                        hit = match & ((inc + mcnt) == r)
                        lv = jnp.maximum(lv, jnp.max(jnp.where(hit, gpos, -1)))
                        return (mcnt + jnp.sum(mi), lv)
                    return lax.fori_loop(0, nv, vec_l, carry)
                _, lv = lax.fori_loop(0, nchunks, chunk_l,
                                      (jnp.int32(0), jnp.int32(-1)))
                return lv
            L = lax.cond(r < E, find_l, lambda _: jnp.int32(n), 0)

            tq = jnp.where(c > 0, T, jnp.int32(0x3FFFFFFF))
            lq = jnp.where(c > 0, L, jnp.int32(-1))
            row_v[...] = jnp.where(lanes == 0, tq * 2,
                                   jnp.where(lanes == 1, lq, 0))
            pltpu.sync_copy(row_v, params_hbm.at[g])

    phase1 = functools.partial(
        pl.kernel,
        out_type=jax.ShapeDtypeStruct((G, 16), jnp.int32),
        mesh=mesh,
        compiler_params=cparams,
        scratch_types=[
            pltpu.VMEM((nsamp_pad,), jnp.int32),
            pltpu.VMEM((win,), jnp.int32),
            pltpu.VMEM((cap,), jnp.int32),
            pltpu.VMEM((16 * 1024,), jnp.int32),
            pltpu.VMEM((1024,), jnp.int32),
            pltpu.VMEM((1024,), jnp.int32),
            pltpu.VMEM((16,), jnp.int32),
            pltpu.SemaphoreType.DMA,
        ],
    )(params_body)

    def mask_body(bits_hbm, eb_hbm, params_hbm, y_hbm,
                  par_v, bits_v, eb_v, out_v, dsem):
        lanes = lax.iota(jnp.int32, 16)
        zer = lanes * 0
        wid = lax.axis_index("s") * NC + lax.axis_index("c")
        pltpu.sync_copy(params_hbm, par_v)
        start = wid * chunk
        nblocks = chunk // blk

        def fire(b, par):
            off = pl.multiple_of(start + b * blk, 8)
            pltpu.make_async_copy(bits_hbm.at[pl.ds(off, blk)],
                                  bits_v.at[pl.ds(par * blk, blk)],
                                  dsem).start()
            pltpu.make_async_copy(eb_hbm.at[pl.ds(off, blk)],
                                  eb_v.at[pl.ds(par * blk, blk)],
                                  dsem).start()

        def drain(par):
            pltpu.make_async_copy(bits_hbm.at[pl.ds(0, blk)],
                                  bits_v.at[pl.ds(par * blk, blk)],
                                  dsem).wait()
            pltpu.make_async_copy(eb_hbm.at[pl.ds(0, blk)],
                                  eb_v.at[pl.ds(par * blk, blk)],
                                  dsem).wait()

        fire(0, 0)

        def blk_body(b, _):
            par = b & 1
            drain(par)

            @pl.when(b + 1 < nblocks)
            def _():
                fire(b + 1, (b + 1) & 1)

            off = pl.multiple_of(start + b * blk, 8)

            def vec_body(i, _):
                v = bits_v[pl.ds(par * blk + i * 16, 16)]
                sg = eb_v[pl.ds(par * blk + i * 16, 16)]
                t2 = plsc.load_gather(par_v, [sg, zer])
                ll = plsc.load_gather(par_v, [sg, zer + 1])
                gpos = off + i * 16 + lanes
                key = (v * 2) | (gpos <= ll).astype(jnp.int32)
                out_v[pl.ds(i * 16, 16)] = jnp.where(
                    key > t2, jnp.float32(1), jnp.float32(0))
                return 0
            lax.fori_loop(0, nvec, vec_body, 0)
            pltpu.sync_copy(out_v, y_hbm.at[pl.ds(off, blk)])
            return 0
        lax.fori_loop(0, nblocks, blk_body, 0)

    phase2 = functools.partial(
        pl.kernel,
        out_type=jax.ShapeDtypeStruct((n,), jnp.float32),
        mesh=mesh,
        compiler_params=cparams,
        scratch_types=[
            pltpu.VMEM((G, 16), jnp.int32),
            pltpu.VMEM((2 * blk,), jnp.int32),
            pltpu.VMEM((2 * blk,), jnp.int32),
            pltpu.VMEM((blk,), jnp.float32),
            pltpu.SemaphoreType.DMA,
        ],
    )(mask_body)

    def run(probs, edge_batch):
        bits = lax.bitcast_convert_type(probs, jnp.int32)
        bits_pad = jnp.pad(bits, (0, pad))
        eb_pad = jnp.pad(edge_batch.astype(jnp.int32), (0, pad),
                         constant_values=BIG)
        samp = eb_pad[: n : stride]
        samp_pad = jnp.pad(samp, (0, nsamp_pad - nsamp), constant_values=BIG)
        params = phase1(bits_pad, eb_pad, samp_pad)
        return phase2(bits_pad, eb_pad, params)

    run.phase1 = phase1
    run.phase2 = phase2
    return run


_run = _make(N, blk=2000, stride=512, cap=48000)


def kernel(probs, edge_batch):
    return _run(probs, edge_batch)


# speedup vs baseline: 189.2564x; 1.4107x over previous
import functools

import jax
import jax.numpy as jnp
from jax import lax
from jax.experimental import pallas as pl
from jax.experimental.pallas import tpu as pltpu
from jax.experimental.pallas import tpu_sc as plsc

N = 1600000
G = 128
NC = 2
NS = 16
NW = NC * NS
SEGS_PER_W = G // NW
BIG = (1 << 30)
SENT = 0x40000000


def _make(n, blk, stride, cap):
    chunk = n // NW
    assert chunk % blk == 0 and blk % 16 == 0 and n % NW == 0
    assert n % stride == 0 and cap % blk == 0
    nsamp = n // stride
    nsamp_pad = ((nsamp + 15) // 16) * 16
    win = ((stride + 1 + 15) // 16) * 16
    pad = blk + win
    nvec = blk // 16
    UNR = 4

    mesh = plsc.VectorSubcoreMesh(core_axis_name="c", subcore_axis_name="s",
                                  num_cores=NC, num_subcores=NS)
    cparams = pltpu.CompilerParams(needs_layout_passes=False)

    def params_body(bits_hbm, eb_hbm, samp_hbm, params_hbm,
                    samp_v, win_v, big_v, hist_v, tot_v, csuf_v, row_v, dsem):
        lanes = lax.iota(jnp.int32, 16)
        wid = lax.axis_index("s") * NC + lax.axis_index("c")
        pltpu.sync_copy(samp_hbm, samp_v)

        nsv = nsamp_pad // 16
        assert nsv % 4 == 0

        def lower_bound(t):
            def cnt_body(i, acc):
                for u in range(4):
                    v = samp_v[pl.ds((i * 4 + u) * 16, 16)]
                    acc = acc + (v < t).astype(jnp.int32)
                return acc
            accv = lax.fori_loop(0, nsv // 4, cnt_body,
                                 jnp.zeros(16, jnp.int32))
            j0 = jnp.sum(accv)
            w0 = pl.multiple_of(jnp.maximum(j0 - 1, 0) * stride, 8)
            pltpu.sync_copy(eb_hbm.at[pl.ds(w0, win)], win_v)

            def cnt2_body(i, acc):
                v = win_v[pl.ds(i * 16, 16)]
                pos = i * 16 + lanes
                m = (pos <= stride) & (v < t)
                return acc + m.astype(jnp.int32)
            accv2 = lax.fori_loop(0, win // 16, cnt2_body,
                                  jnp.zeros(16, jnp.int32))
            return w0 + jnp.sum(accv2)

        bounds = [lower_bound(SEGS_PER_W * wid + j)
                  for j in range(SEGS_PER_W + 1)]

        for j in range(SEGS_PER_W):
            g = SEGS_PER_W * wid + j
            s, e = bounds[j], bounds[j + 1]
            c = e - s
            m = (c - 1) >> 1
            k0 = jnp.where((c & 1) == 0, c >> 1, m + (m & 1))
            k = jnp.minimum(jnp.maximum(k0, 1), c)
            base = s & jnp.int32(-8)
            head = s - base
            c_adj = c + head
            nchunks = (c_adj + cap - 1) // cap

            def load_chunk(ch):
                clen = jnp.minimum(c_adj - ch * cap, cap)
                nb = (clen + blk - 1) // blk

                def fire(b, _):
                    off = pl.multiple_of(base + ch * cap + b * blk, 8)
                    pltpu.make_async_copy(
                        bits_hbm.at[pl.ds(off, blk)],
                        big_v.at[pl.ds(b * blk, blk)], dsem).start()
                    return 0
                lax.fori_loop(0, nb, fire, 0)

                def drain(b, _):
                    pltpu.make_async_copy(
                        bits_hbm.at[pl.ds(0, blk)],
                        big_v.at[pl.ds(0, blk)], dsem).wait()
                    return 0
                lax.fori_loop(0, nb, drain, 0)
                head_eff = jnp.where(ch == 0, head, 0)
                v0 = big_v[pl.ds(0, 16)]
                big_v[pl.ds(0, 16)] = jnp.where(lanes < head_eff, SENT, v0)
                nv = (clen + 15) // 16
                toff = (nv - 1) * 16
                vt = big_v[pl.ds(toff, 16)]
                big_v[pl.ds(toff, 16)] = jnp.where(
                    toff + lanes < clen, vt, SENT)
                sentv = jnp.zeros(16, jnp.int32) + SENT
                for u in range(UNR - 1):
                    big_v[pl.ds((nv + u) * 16, 16)] = sentv
                return nv

            def radix_pass(p, carry):
                prefix, k_rem, _ = carry
                shift = 20 - 10 * p

                def zero_body(i, _):
                    z = jnp.zeros(16, jnp.int32)
                    for u in range(8):
                        hist_v[pl.ds((i * 8 + u) * 16, 16)] = z
                    return 0
                lax.fori_loop(0, 128, zero_body, 0)

                def chunk_body(ch, _):
                    clen = jnp.minimum(c_adj - ch * cap, cap)
                    nv = (clen + 15) // 16

                    @pl.when((p == 0) | (nchunks > 1))
                    def _():
                        load_chunk(ch)

                    ones = jnp.ones(16, jnp.int32)
                    lidx = lanes * 1024

                    def vec_body(i, _):
                        for u in range(UNR):
                            v = big_v[pl.ds((i * UNR + u) * 16, 16)]
                            part = (v >> (shift + 10)) == prefix
                            dig = (v >> shift) & 1023
                            plsc.addupdate_scatter(
                                hist_v, [lidx + dig], ones, mask=part)
                        return 0
                    lax.fori_loop(0, (nv + UNR - 1) // UNR, vec_body, 0)
                    return 0
                lax.fori_loop(0, nchunks, chunk_body, 0)

                def red_body(chh, _):
                    acc = hist_v[pl.ds(chh * 16, 16)]
                    for l in range(1, 16):
                        acc = acc + hist_v[pl.ds(l * 1024 + chh * 16, 16)]
                    tot_v[pl.ds(chh * 16, 16)] = acc
                    return 0
                lax.fori_loop(0, 64, red_body, 0)

                def s_body(t, carry2):
                    running, accnt = carry2
                    jv = 63 - t
                    v = tot_v[pl.ds(jv * 16, 16)]
                    ssum = jnp.sum(v)
                    pe = plsc.cumsum(v) - v
                    cv = running + (ssum - pe)
                    csuf_v[pl.ds(jv * 16, 16)] = cv
                    accnt = accnt + (cv >= k_rem).astype(jnp.int32)
                    return (running + ssum, accnt)
                _, accnt = lax.fori_loop(
                    0, 64, s_body, (jnp.int32(0), jnp.zeros(16, jnp.int32)))
                dstar = jnp.sum(accnt) - 1
                didx = jnp.zeros(16, jnp.int32) + dstar
                totd = jnp.max(plsc.load_gather(tot_v, [didx]))
                cd = jnp.max(plsc.load_gather(csuf_v, [didx]))
                k_rem = k_rem - (cd - totd)
                prefix = (prefix << 10) | dstar
                return (prefix, k_rem, totd)

            T, r, E = lax.fori_loop(
                0, 3, radix_pass, (jnp.int32(0), k, jnp.int32(0)))

            def find_l(_):
                def chunk_l(ch, carry):
                    clen = jnp.minimum(c_adj - ch * cap, cap)
                    nv = (clen + 15) // 16

                    @pl.when(nchunks > 1)
                    def _():
                        load_chunk(ch)

                    def vec_l(i, carry2):
                        mcnt, lv = carry2
                        v = big_v[pl.ds(i * 16, 16)]
                        gpos = base + ch * cap + i * 16 + lanes
                        match = v == T
                        mi = match.astype(jnp.int32)
                        inc = plsc.cumsum(mi)
                        hit = match & ((inc + mcnt) == r)
                        lv = jnp.maximum(lv, jnp.max(jnp.where(hit, gpos, -1)))
                        return (mcnt + jnp.sum(mi), lv)
                    return lax.fori_loop(0, nv, vec_l, carry)
                _, lv = lax.fori_loop(0, nchunks, chunk_l,
                                      (jnp.int32(0), jnp.int32(-1)))
                return lv
            L = lax.cond(r < E, find_l, lambda _: jnp.int32(n), 0)

            tq = jnp.where(c > 0, T, jnp.int32(0x3FFFFFFF))
            lq = jnp.where(c > 0, L, jnp.int32(-1))
            row_v[...] = jnp.where(lanes == 0, tq * 2,
                                   jnp.where(lanes == 1, lq, 0))
            pltpu.sync_copy(row_v, params_hbm.at[g])

    phase1 = functools.partial(
        pl.kernel,
        out_type=jax.ShapeDtypeStruct((G, 16), jnp.int32),
        mesh=mesh,
        compiler_params=cparams,
        scratch_types=[
            pltpu.VMEM((nsamp_pad,), jnp.int32),
            pltpu.VMEM((win,), jnp.int32),
            pltpu.VMEM((cap + 16 * UNR,), jnp.int32),
            pltpu.VMEM((16 * 1024,), jnp.int32),
            pltpu.VMEM((1024,), jnp.int32),
            pltpu.VMEM((1024,), jnp.int32),
            pltpu.VMEM((16,), jnp.int32),
            pltpu.SemaphoreType.DMA,
        ],
    )(params_body)

    def mask_body(bits_hbm, eb_hbm, params_hbm, y_hbm,
                  par_v, bits_v, eb_v, out_v, dsem, osem):
        lanes = lax.iota(jnp.int32, 16)
        zer = lanes * 0
        wid = lax.axis_index("s") * NC + lax.axis_index("c")
        pltpu.sync_copy(params_hbm, par_v)
        start = wid * chunk
        nblocks = chunk // blk

        def fire(b, par):
            off = pl.multiple_of(start + b * blk, 8)
            pltpu.make_async_copy(bits_hbm.at[pl.ds(off, blk)],
                                  bits_v.at[pl.ds(par * blk, blk)],
                                  dsem).start()
            pltpu.make_async_copy(eb_hbm.at[pl.ds(off, blk)],
                                  eb_v.at[pl.ds(par * blk, blk)],
                                  dsem).start()

        def drain(par):
            pltpu.make_async_copy(bits_hbm.at[pl.ds(0, blk)],
                                  bits_v.at[pl.ds(par * blk, blk)],
                                  dsem).wait()
            pltpu.make_async_copy(eb_hbm.at[pl.ds(0, blk)],
                                  eb_v.at[pl.ds(par * blk, blk)],
                                  dsem).wait()

        def out_drain():
            pltpu.make_async_copy(out_v.at[pl.ds(0, blk)],
                                  y_hbm.at[pl.ds(0, blk)], osem).wait()

        fire(0, 0)

        def blk_body(b, _):
            par = b & 1
            obase = par * blk
            drain(par)

            @pl.when(b + 1 < nblocks)
            def _():
                fire(b + 1, (b + 1) & 1)

            @pl.when(b >= 2)
            def _():
                out_drain()

            off = pl.multiple_of(start + b * blk, 8)
            sg0 = eb_v[pl.ds(par * blk, 16)]
            sgl = eb_v[pl.ds(par * blk + blk - 16, 16)]
            first = jnp.min(sg0)
            last = jnp.max(sgl)

            def fast(_):
                tidx = zer + first
                t2s = jnp.max(plsc.load_gather(par_v, [tidx, zer]))
                lls = jnp.max(plsc.load_gather(par_v, [tidx, zer + 1]))

                def vec_f(i, _):
                    v = bits_v[pl.ds(par * blk + i * 16, 16)]
                    gpos = off + i * 16 + lanes
                    key = (v * 2) | (gpos <= lls).astype(jnp.int32)
                    out_v[pl.ds(obase + i * 16, 16)] = jnp.where(
                        key > t2s, jnp.float32(1), jnp.float32(0))
                    return 0
                lax.fori_loop(0, nvec, vec_f, 0, unroll=5)
                return 0

            def slow(_):
                def vec_body(i, _):
                    v = bits_v[pl.ds(par * blk + i * 16, 16)]
                    sg = eb_v[pl.ds(par * blk + i * 16, 16)]
                    t2 = plsc.load_gather(par_v, [sg, zer])
                    ll = plsc.load_gather(par_v, [sg, zer + 1])
                    gpos = off + i * 16 + lanes
                    key = (v * 2) | (gpos <= ll).astype(jnp.int32)
                    out_v[pl.ds(obase + i * 16, 16)] = jnp.where(
                        key > t2, jnp.float32(1), jnp.float32(0))
                    return 0
                lax.fori_loop(0, nvec, vec_body, 0, unroll=5)
                return 0

            lax.cond(first == last, fast, slow, 0)
            pltpu.make_async_copy(out_v.at[pl.ds(obase, blk)],
                                  y_hbm.at[pl.ds(off, blk)], osem).start()
            return 0
        lax.fori_loop(0, nblocks, blk_body, 0)
        out_drain()
        out_drain()

    phase2 = functools.partial(
        pl.kernel,
        out_type=jax.ShapeDtypeStruct((n,), jnp.float32),
        mesh=mesh,
        compiler_params=cparams,
        scratch_types=[
            pltpu.VMEM((G, 16), jnp.int32),
            pltpu.VMEM((2 * blk,), jnp.int32),
            pltpu.VMEM((2 * blk,), jnp.int32),
            pltpu.VMEM((2 * blk,), jnp.float32),
            pltpu.SemaphoreType.DMA,
            pltpu.SemaphoreType.DMA,
        ],
    )(mask_body)

    def run(probs, edge_batch):
        bits = lax.bitcast_convert_type(probs, jnp.int32)
        bits_pad = jnp.pad(bits, (0, pad))
        eb_pad = jnp.pad(edge_batch.astype(jnp.int32), (0, pad),
                         constant_values=BIG)
        samp = eb_pad[: n : stride]
        samp_pad = jnp.pad(samp, (0, nsamp_pad - nsamp), constant_values=BIG)
        params = phase1(bits_pad, eb_pad, samp_pad)
        return phase2(bits_pad, eb_pad, params)

    run.phase1 = phase1
    run.phase2 = phase2
    return run


_run = _make(N, blk=2000, stride=512, cap=48000)


def kernel(probs, edge_batch):
    return _run(probs, edge_batch)
